# baseline (speedup 1.0000x reference)
import functools

import numpy as np
import jax
import jax.numpy as jnp
from jax.experimental import pallas as pl
from jax.experimental.pallas import tpu as pltpu

_BM = 256
_BC = 640


def _exact_mean(v):
    n = v.shape[0]
    size = 1
    while size < n:
        size *= 2
    x = jnp.pad(v, (0, size - n))
    err_total = jnp.float32(0.0)
    while size > 1:
        h = size // 2
        a, b = x[:h], x[h:]
        s = a + b
        bv = s - a
        err = (a - (s - bv)) + (b - bv)
        err_total = err_total + jnp.sum(err)
        x = s
        size = h
    return (x[0] + err_total) / n


def _gcn_body(adj_ref, fa_ref, wcat_ref, aw1t_ref, ab1_ref, aw2t_ref, avec_ref,
              z_ref, w_ref, seq_ref, *, n_valid, bm):
    m = pl.program_id(0)

    @pl.when(m == 0)
    def _init():
        seq_ref[...] = jnp.dot(fa_ref[...], wcat_ref[...],
                               preferred_element_type=jnp.float32
                               ).astype(jnp.bfloat16)
        w_ref[0] = 0.0
        w_ref[1] = 0.0

    adj_b = adj_ref[...].astype(jnp.bfloat16)
    h = jnp.dot(adj_b, seq_ref[...], preferred_element_type=jnp.float32)
    z = jnp.where(h >= 0.0, h, avec_ref[...] * h)
    z_ref[...] = z
    rows = m * bm + jax.lax.broadcasted_iota(jnp.int32, (bm, 1), 0)
    hh = z.shape[1] // 2
    for l in range(2):
        zl = z[:, l * hh:(l + 1) * hh]
        t = jnp.tanh(jnp.dot(zl, aw1t_ref[...],
                             preferred_element_type=jnp.float32) + ab1_ref[...])
        s = jnp.dot(t, aw2t_ref[...], preferred_element_type=jnp.float32)
        s = jnp.where(rows < n_valid, s, 0.0)
        w_ref[l] = w_ref[l] + jnp.sum(s)


def _proj_body(z_ref, f_ref, beta_ref, lwt_ref,
               mw1t_ref, mb1_ref, mw2t_ref, mb2_ref,
               xw1t_ref, xb1_ref, xw2t_ref, xb2_ref,
               amin_ref, bmin_ref, amax_ref, bmax_ref,
               *, n_valid, bc, inv_tau):
    hh = z_ref.shape[1] // 2
    z_mp = z_ref[:, :hh] * beta_ref[0] + z_ref[:, hh:] * beta_ref[1]
    z_sc = jnp.dot(f_ref[...], lwt_ref[...], preferred_element_type=jnp.float32)
    rows = pl.program_id(0) * bc + jax.lax.broadcasted_iota(
        jnp.int32, (bc, 1), 0)
    valid = rows < n_valid

    def proj(x, w1t, b1, w2t, b2, scale):
        t = jnp.dot(x, w1t, preferred_element_type=jnp.float32) + b1
        t = jnp.where(t > 0.0, t, jnp.exp(t) - 1.0)
        p = jnp.dot(t, w2t, preferred_element_type=jnp.float32) + b2
        nrm = jnp.sqrt(jnp.sum(p * p, axis=1, keepdims=True))
        p = p * (scale / jnp.maximum(nrm, 1e-12))
        return jnp.where(valid, p, 0.0).astype(jnp.bfloat16)

    amin_ref[...] = proj(z_mp, mw1t_ref[...], mb1_ref[...], mw2t_ref[...],
                         mb2_ref[...], inv_tau)
    bmin_ref[...] = proj(z_sc, mw1t_ref[...], mb1_ref[...], mw2t_ref[...],
                         mb2_ref[...], 1.0)
    amax_ref[...] = proj(z_mp, xw1t_ref[...], xb1_ref[...], xw2t_ref[...],
                         xb2_ref[...], inv_tau)
    bmax_ref[...] = proj(z_sc, xw1t_ref[...], xb1_ref[...], xw2t_ref[...],
                         xb2_ref[...], 1.0)


def _sweep_body(ii_ref, jj_ref, amin_ref, bmin_ref, amax_ref, bmax_ref,
                pij_ref, pji_ref, qij_ref, qji_ref, out_ref, *, n_valid, b,
                masked):
    p = pl.program_id(0)
    bi = ii_ref[p]
    bj = jj_ref[p]
    i0 = bi * b
    j0 = bj * b
    nd = jnp.where(bi == bj, 0.0, 1.0)
    if masked:
        rows = jax.lax.broadcasted_iota(jnp.int32, (b, b), 0)
        cols = jax.lax.broadcasted_iota(jnp.int32, (b, b), 1)
        v_ij = ((i0 + rows) < n_valid) & ((j0 + cols) < n_valid)
        v_ji = ((j0 + rows) < n_valid) & ((i0 + cols) < n_valid)
        inv_i = jnp.maximum(i0 + b - n_valid, 0).astype(jnp.float32)
        inv_j = jnp.maximum(j0 + b - n_valid, 0).astype(jnp.float32)
        pijT = jnp.where(v_ij, pij_ref[...], 0.0).T
        pjiT = jnp.where(v_ji, pji_ref[...], 0.0).T
        qijT = jnp.where(v_ij, qij_ref[...], 0.0).T
        qjiT = jnp.where(v_ji, qji_ref[...], 0.0).T
    else:
        inv_i = inv_j = 0.0
        pijT = pij_ref[...].T
        pjiT = pji_ref[...].T
        qijT = qij_ref[...].T
        qjiT = qji_ref[...].T

    def half(a, b2, pos_t_row, pos_t_col, inv_r, inv_c):
        s = jax.lax.dot_general(a, b2, (((1,), (1,)), ((), ())),
                                preferred_element_type=jnp.float32)
        st = jax.lax.dot_general(b2, a, (((1,), (1,)), ((), ())),
                                 preferred_element_type=jnp.float32)
        m = jnp.exp(s)
        mt = jnp.exp(st)
        rowsum = jnp.sum(mt, axis=0) - inv_c
        colsum = jnp.sum(m, axis=0) - inv_r
        rowdot = jnp.sum(mt * pos_t_row, axis=0)
        coldot = jnp.sum(m * pos_t_col, axis=0)
        return rowsum, colsum, rowdot, coldot

    for c, (a_ref, b_ref, dijT, djiT) in enumerate((
            (amin_ref, bmin_ref, pijT, pjiT),
            (amax_ref, bmax_ref, qijT, qjiT))):
        a_i = a_ref[pl.ds(i0, b), :]
        a_j = a_ref[pl.ds(j0, b), :]
        b_i = b_ref[pl.ds(i0, b), :]
        b_j = b_ref[pl.ds(j0, b), :]
        rsI, csJ, rdI, cdJ = half(a_i, b_j, dijT, djiT, inv_i, inv_j)
        rsJ, csI, rdJ, cdI = half(a_j, b_i, djiT, dijT, inv_j, inv_i)
        o = 4 * c
        out_ref[0, o + 0, :] = rsI
        out_ref[0, o + 1, :] = csI * nd
        out_ref[0, o + 2, :] = rdI
        out_ref[0, o + 3, :] = cdI * nd
        out_ref[0, o + 8, :] = rsJ * nd
        out_ref[0, o + 9, :] = csJ
        out_ref[0, o + 10, :] = rdJ * nd
        out_ref[0, o + 11, :] = cdJ


def kernel(adj, features, pos, pos_outer, gcn_W, prelu_a, att_W1, att_b1,
           att_W2, min_W1, min_b1, min_W2, min_b2, max_W1, max_b1, max_W2,
           max_b2, l_W):
    tau, lam = 0.8, 0.5
    n, d = features.shape
    hdim = gcn_W.shape[1]
    t_blk = pl.cdiv(n, _BC)
    npad = t_blk * _BC
    assert npad % _BM == 0
    m_blk = npad // _BM

    noise = jax.random.normal(jax.random.key(42), features.shape,
                              features.dtype) * 0.01
    nn = jnp.linalg.norm(noise, axis=1, keepdims=True)
    fa = features + noise / jnp.maximum(nn, 1e-12)
    wcat = jnp.concatenate([gcn_W[0].T, gcn_W[1].T], axis=1)
    avec = jnp.concatenate(
        [jnp.full((1, hdim), prelu_a[0], jnp.float32),
         jnp.full((1, hdim), prelu_a[1], jnp.float32)], axis=1)
    ab1 = att_b1.reshape(1, hdim)
    aw2t = att_W2.T

    z, wsum = pl.pallas_call(
        functools.partial(_gcn_body, n_valid=n, bm=_BM),
        grid=(m_blk,),
        in_specs=[
            pl.BlockSpec((_BM, n), lambda m: (m, 0)),
            pl.BlockSpec((n, d), lambda m: (0, 0)),
            pl.BlockSpec((d, 2 * hdim), lambda m: (0, 0)),
            pl.BlockSpec((hdim, hdim), lambda m: (0, 0)),
            pl.BlockSpec((1, hdim), lambda m: (0, 0)),
            pl.BlockSpec((hdim, 1), lambda m: (0, 0)),
            pl.BlockSpec((1, 2 * hdim), lambda m: (0, 0)),
        ],
        out_specs=[
            pl.BlockSpec((_BM, 2 * hdim), lambda m: (m, 0)),
            pl.BlockSpec(memory_space=pltpu.SMEM),
        ],
        out_shape=[
            jax.ShapeDtypeStruct((npad, 2 * hdim), jnp.float32),
            jax.ShapeDtypeStruct((2,), jnp.float32),
        ],
        scratch_shapes=[pltpu.VMEM((n, 2 * hdim), jnp.bfloat16)],
        compiler_params=pltpu.CompilerParams(
            dimension_semantics=("arbitrary",)),
    )(adj, fa, wcat, att_W1.T, ab1, aw2t, avec)

    beta = jax.nn.softmax(wsum / n)

    amin, bmin, amax, bmax = pl.pallas_call(
        functools.partial(_proj_body, n_valid=n, bc=_BC, inv_tau=1.0 / tau),
        grid=(t_blk,),
        in_specs=[
            pl.BlockSpec((_BC, 2 * hdim), lambda m: (m, 0)),
            pl.BlockSpec((_BC, d), lambda m: (m, 0)),
            pl.BlockSpec(memory_space=pltpu.SMEM),
            pl.BlockSpec((d, hdim), lambda m: (0, 0)),
            pl.BlockSpec((hdim, hdim), lambda m: (0, 0)),
            pl.BlockSpec((1, hdim), lambda m: (0, 0)),
            pl.BlockSpec((hdim, hdim), lambda m: (0, 0)),
            pl.BlockSpec((1, hdim), lambda m: (0, 0)),
            pl.BlockSpec((hdim, hdim), lambda m: (0, 0)),
            pl.BlockSpec((1, hdim), lambda m: (0, 0)),
            pl.BlockSpec((hdim, hdim), lambda m: (0, 0)),
            pl.BlockSpec((1, hdim), lambda m: (0, 0)),
        ],
        out_specs=[pl.BlockSpec((_BC, hdim), lambda m: (m, 0))] * 4,
        out_shape=[jax.ShapeDtypeStruct((npad, hdim), jnp.bfloat16)] * 4,
        compiler_params=pltpu.CompilerParams(
            dimension_semantics=("arbitrary",)),
    )(z, features, beta, l_W.T,
      min_W1.T, min_b1.reshape(1, hdim), min_W2.T, min_b2.reshape(1, hdim),
      max_W1.T, max_b1.reshape(1, hdim), max_W2.T, max_b2.reshape(1, hdim))

    all_pairs = [(i, j) for i in range(t_blk) for j in range(i, t_blk)]
    ragged = (n % _BC) != 0
    edge_pairs = [q for q in all_pairs
                  if ragged and (q[0] == t_blk - 1 or q[1] == t_blk - 1)]
    int_pairs = [q for q in all_pairs if q not in edge_pairs]

    def sweep(pair_list, masked_flag):
        ii = jnp.asarray(np.array([q[0] for q in pair_list], np.int32))
        jj = jnp.asarray(np.array([q[1] for q in pair_list], np.int32))
        cnt = len(pair_list)
        parts = pl.pallas_call(
            functools.partial(_sweep_body, n_valid=n, b=_BC,
                              masked=masked_flag),
            grid_spec=pltpu.PrefetchScalarGridSpec(
                num_scalar_prefetch=2,
                grid=(cnt,),
                in_specs=[
                    pl.BlockSpec((npad, hdim), lambda p, ii, jj: (0, 0)),
                    pl.BlockSpec((npad, hdim), lambda p, ii, jj: (0, 0)),
                    pl.BlockSpec((npad, hdim), lambda p, ii, jj: (0, 0)),
                    pl.BlockSpec((npad, hdim), lambda p, ii, jj: (0, 0)),
                    pl.BlockSpec((_BC, _BC), lambda p, ii, jj: (ii[p], jj[p])),
                    pl.BlockSpec((_BC, _BC), lambda p, ii, jj: (jj[p], ii[p])),
                    pl.BlockSpec((_BC, _BC), lambda p, ii, jj: (ii[p], jj[p])),
                    pl.BlockSpec((_BC, _BC), lambda p, ii, jj: (jj[p], ii[p])),
                ],
                out_specs=pl.BlockSpec((1, 16, _BC),
                                       lambda p, ii, jj: (p, 0, 0)),
            ),
            out_shape=jax.ShapeDtypeStruct((cnt, 16, _BC), jnp.float32),
            compiler_params=pltpu.CompilerParams(
                dimension_semantics=("arbitrary",)),
        )(ii, jj, amin, bmin, amax, bmax, pos, pos, pos_outer, pos_outer)
        return ii, jj, parts

    acc = jnp.zeros((t_blk, 8, _BC), jnp.float32)
    for plist, mflag in ((int_pairs, False), (edge_pairs, True)):
        if plist:
            ii, jj, parts = sweep(plist, mflag)
            acc = acc.at[ii].add(parts[:, 0:8, :]).at[jj].add(
                parts[:, 8:16, :])
    stats = acc.transpose(1, 0, 2).reshape(8, npad)[:, :n]
    rs_min, cs_min, rd_min, cd_min, rs_max, cs_max, rd_max, cd_max = stats

    eps = 1e-8
    lori_mp = -_exact_mean(jnp.log(rd_min / (rs_min + eps)))
    lori_sc = -_exact_mean(jnp.log(cd_min / (cs_min + eps)))
    loss_min = lam * lori_mp + (1.0 - lam) * lori_sc
    l1 = _exact_mean(jnp.log(rd_max / (rs_max + eps)))
    l2 = _exact_mean(jnp.log(cd_max / (cs_max + eps)))
    loss_max = (l1 + l2) / 2.0
    return loss_min + loss_max


# speedup vs baseline: 1.1196x; 1.1196x over previous
import functools

import numpy as np
import jax
import jax.numpy as jnp
from jax.experimental import pallas as pl
from jax.experimental.pallas import tpu as pltpu

_BM = 256
_BC = 640


def _exact_means(v):
    r, n = v.shape
    size = 1
    while size < n:
        size *= 2
    x = jnp.pad(v, ((0, 0), (0, size - n)))
    err_total = jnp.zeros((r,), jnp.float32)
    while size > 1:
        h = size // 2
        a, b = x[:, :h], x[:, h:]
        s = a + b
        bv = s - a
        err = (a - (s - bv)) + (b - bv)
        err_total = err_total + jnp.sum(err, axis=1)
        x = s
        size = h
    return (x[:, 0] + err_total) / n


def _gcn_body(adj_ref, fa_ref, wcat_ref, aw1t_ref, ab1_ref, aw2t_ref, avec_ref,
              z_ref, w_ref, seq_ref, *, n_valid, bm):
    m = pl.program_id(0)

    @pl.when(m == 0)
    def _init():
        seq_ref[...] = jnp.dot(fa_ref[...], wcat_ref[...],
                               preferred_element_type=jnp.float32
                               ).astype(jnp.bfloat16)
        w_ref[0] = 0.0
        w_ref[1] = 0.0

    adj_b = adj_ref[...].astype(jnp.bfloat16)
    h = jnp.dot(adj_b, seq_ref[...], preferred_element_type=jnp.float32)
    z = jnp.where(h >= 0.0, h, avec_ref[...] * h)
    z_ref[...] = z
    rows = m * bm + jax.lax.broadcasted_iota(jnp.int32, (bm, 1), 0)
    hh = z.shape[1] // 2
    for l in range(2):
        zl = z[:, l * hh:(l + 1) * hh]
        t = jnp.tanh(jnp.dot(zl, aw1t_ref[...],
                             preferred_element_type=jnp.float32) + ab1_ref[...])
        s = jnp.dot(t, aw2t_ref[...], preferred_element_type=jnp.float32)
        s = jnp.where(rows < n_valid, s, 0.0)
        w_ref[l] = w_ref[l] + jnp.sum(s)


def _proj_body(z_ref, f_ref, beta_ref, lwt_ref,
               mw1t_ref, mb1_ref, mw2t_ref, mb2_ref,
               xw1t_ref, xb1_ref, xw2t_ref, xb2_ref,
               amin_ref, bmin_ref, amax_ref, bmax_ref,
               *, n_valid, bc, inv_tau):
    hh = z_ref.shape[1] // 2
    z_mp = z_ref[:, :hh] * beta_ref[0] + z_ref[:, hh:] * beta_ref[1]
    z_sc = jnp.dot(f_ref[...], lwt_ref[...], preferred_element_type=jnp.float32)
    rows = pl.program_id(0) * bc + jax.lax.broadcasted_iota(
        jnp.int32, (bc, 1), 0)
    valid = rows < n_valid

    def proj(x, w1t, b1, w2t, b2, scale):
        t = jnp.dot(x, w1t, preferred_element_type=jnp.float32) + b1
        t = jnp.where(t > 0.0, t, jnp.exp(t) - 1.0)
        p = jnp.dot(t, w2t, preferred_element_type=jnp.float32) + b2
        nrm = jnp.sqrt(jnp.sum(p * p, axis=1, keepdims=True))
        p = p * (scale / jnp.maximum(nrm, 1e-12))
        return jnp.where(valid, p, 0.0).astype(jnp.bfloat16)

    amin_ref[...] = proj(z_mp, mw1t_ref[...], mb1_ref[...], mw2t_ref[...],
                         mb2_ref[...], inv_tau)
    bmin_ref[...] = proj(z_sc, mw1t_ref[...], mb1_ref[...], mw2t_ref[...],
                         mb2_ref[...], 1.0)
    amax_ref[...] = proj(z_mp, xw1t_ref[...], xb1_ref[...], xw2t_ref[...],
                         xb2_ref[...], inv_tau)
    bmax_ref[...] = proj(z_sc, xw1t_ref[...], xb1_ref[...], xw2t_ref[...],
                         xb2_ref[...], 1.0)


def _sweep_body(ii_ref, jj_ref, amin_ref, bmin_ref, amax_ref, bmax_ref,
                pij_ref, pji_ref, qij_ref, qji_ref, out_ref, *, n_valid, b,
                masked):
    p = pl.program_id(0)
    bi = ii_ref[p]
    bj = jj_ref[p]
    i0 = bi * b
    j0 = bj * b
    nd = jnp.where(bi == bj, 0.0, 1.0)
    if masked:
        rows = jax.lax.broadcasted_iota(jnp.int32, (b, b), 0)
        cols = jax.lax.broadcasted_iota(jnp.int32, (b, b), 1)
        v_ij = ((i0 + rows) < n_valid) & ((j0 + cols) < n_valid)
        v_ji = ((j0 + rows) < n_valid) & ((i0 + cols) < n_valid)
        inv_i = jnp.maximum(i0 + b - n_valid, 0).astype(jnp.float32)
        inv_j = jnp.maximum(j0 + b - n_valid, 0).astype(jnp.float32)
        pijT = jnp.where(v_ij, pij_ref[...], 0.0).T
        pjiT = jnp.where(v_ji, pji_ref[...], 0.0).T
        qijT = jnp.where(v_ij, qij_ref[...], 0.0).T
        qjiT = jnp.where(v_ji, qji_ref[...], 0.0).T
    else:
        inv_i = inv_j = 0.0
        pijT = pij_ref[...].T
        pjiT = pji_ref[...].T
        qijT = qij_ref[...].T
        qjiT = qji_ref[...].T

    def half(a, b2, pos_t_row, pos_t_col, inv_r, inv_c):
        s = jax.lax.dot_general(a, b2, (((1,), (1,)), ((), ())),
                                preferred_element_type=jnp.float32)
        st = jax.lax.dot_general(b2, a, (((1,), (1,)), ((), ())),
                                 preferred_element_type=jnp.float32)
        m = jnp.exp(s)
        mt = jnp.exp(st)
        rowsum = jnp.sum(mt, axis=0) - inv_c
        colsum = jnp.sum(m, axis=0) - inv_r
        rowdot = jnp.sum(mt * pos_t_row, axis=0)
        coldot = jnp.sum(m * pos_t_col, axis=0)
        return rowsum, colsum, rowdot, coldot

    for c, (a_ref, b_ref, dijT, djiT) in enumerate((
            (amin_ref, bmin_ref, pijT, pjiT),
            (amax_ref, bmax_ref, qijT, qjiT))):
        a_i = a_ref[pl.ds(i0, b), :]
        a_j = a_ref[pl.ds(j0, b), :]
        b_i = b_ref[pl.ds(i0, b), :]
        b_j = b_ref[pl.ds(j0, b), :]
        rsI, csJ, rdI, cdJ = half(a_i, b_j, dijT, djiT, inv_i, inv_j)
        rsJ, csI, rdJ, cdI = half(a_j, b_i, djiT, dijT, inv_j, inv_i)
        o = 4 * c
        out_ref[0, o + 0, :] = rsI
        out_ref[0, o + 1, :] = csI * nd
        out_ref[0, o + 2, :] = rdI
        out_ref[0, o + 3, :] = cdI * nd
        out_ref[0, o + 8, :] = rsJ * nd
        out_ref[0, o + 9, :] = csJ
        out_ref[0, o + 10, :] = rdJ * nd
        out_ref[0, o + 11, :] = cdJ


def kernel(adj, features, pos, pos_outer, gcn_W, prelu_a, att_W1, att_b1,
           att_W2, min_W1, min_b1, min_W2, min_b2, max_W1, max_b1, max_W2,
           max_b2, l_W):
    tau, lam = 0.8, 0.5
    n, d = features.shape
    hdim = gcn_W.shape[1]
    t_blk = pl.cdiv(n, _BC)
    npad = t_blk * _BC
    assert npad % _BM == 0
    m_blk = npad // _BM

    noise = jax.random.normal(jax.random.key(42), features.shape,
                              features.dtype) * 0.01
    nn = jnp.linalg.norm(noise, axis=1, keepdims=True)
    fa = features + noise / jnp.maximum(nn, 1e-12)
    wcat = jnp.concatenate([gcn_W[0].T, gcn_W[1].T], axis=1)
    avec = jnp.concatenate(
        [jnp.full((1, hdim), prelu_a[0], jnp.float32),
         jnp.full((1, hdim), prelu_a[1], jnp.float32)], axis=1)
    ab1 = att_b1.reshape(1, hdim)
    aw2t = att_W2.T

    z, wsum = pl.pallas_call(
        functools.partial(_gcn_body, n_valid=n, bm=_BM),
        grid=(m_blk,),
        in_specs=[
            pl.BlockSpec((_BM, n), lambda m: (m, 0)),
            pl.BlockSpec((n, d), lambda m: (0, 0)),
            pl.BlockSpec((d, 2 * hdim), lambda m: (0, 0)),
            pl.BlockSpec((hdim, hdim), lambda m: (0, 0)),
            pl.BlockSpec((1, hdim), lambda m: (0, 0)),
            pl.BlockSpec((hdim, 1), lambda m: (0, 0)),
            pl.BlockSpec((1, 2 * hdim), lambda m: (0, 0)),
        ],
        out_specs=[
            pl.BlockSpec((_BM, 2 * hdim), lambda m: (m, 0)),
            pl.BlockSpec(memory_space=pltpu.SMEM),
        ],
        out_shape=[
            jax.ShapeDtypeStruct((npad, 2 * hdim), jnp.float32),
            jax.ShapeDtypeStruct((2,), jnp.float32),
        ],
        scratch_shapes=[pltpu.VMEM((n, 2 * hdim), jnp.bfloat16)],
        compiler_params=pltpu.CompilerParams(
            dimension_semantics=("arbitrary",)),
    )(adj, fa, wcat, att_W1.T, ab1, aw2t, avec)

    beta = jax.nn.softmax(wsum / n)

    amin, bmin, amax, bmax = pl.pallas_call(
        functools.partial(_proj_body, n_valid=n, bc=_BC, inv_tau=1.0 / tau),
        grid=(t_blk,),
        in_specs=[
            pl.BlockSpec((_BC, 2 * hdim), lambda m: (m, 0)),
            pl.BlockSpec((_BC, d), lambda m: (m, 0)),
            pl.BlockSpec(memory_space=pltpu.SMEM),
            pl.BlockSpec((d, hdim), lambda m: (0, 0)),
            pl.BlockSpec((hdim, hdim), lambda m: (0, 0)),
            pl.BlockSpec((1, hdim), lambda m: (0, 0)),
            pl.BlockSpec((hdim, hdim), lambda m: (0, 0)),
            pl.BlockSpec((1, hdim), lambda m: (0, 0)),
            pl.BlockSpec((hdim, hdim), lambda m: (0, 0)),
            pl.BlockSpec((1, hdim), lambda m: (0, 0)),
            pl.BlockSpec((hdim, hdim), lambda m: (0, 0)),
            pl.BlockSpec((1, hdim), lambda m: (0, 0)),
        ],
        out_specs=[pl.BlockSpec((_BC, hdim), lambda m: (m, 0))] * 4,
        out_shape=[jax.ShapeDtypeStruct((npad, hdim), jnp.bfloat16)] * 4,
        compiler_params=pltpu.CompilerParams(
            dimension_semantics=("arbitrary",)),
    )(z, features, beta, l_W.T,
      min_W1.T, min_b1.reshape(1, hdim), min_W2.T, min_b2.reshape(1, hdim),
      max_W1.T, max_b1.reshape(1, hdim), max_W2.T, max_b2.reshape(1, hdim))

    all_pairs = [(i, j) for i in range(t_blk) for j in range(i, t_blk)]
    ragged = (n % _BC) != 0
    edge_pairs = [q for q in all_pairs
                  if ragged and (q[0] == t_blk - 1 or q[1] == t_blk - 1)]
    int_pairs = [q for q in all_pairs if q not in edge_pairs]

    def sweep(pair_list, masked_flag):
        ii = jnp.asarray(np.array([q[0] for q in pair_list], np.int32))
        jj = jnp.asarray(np.array([q[1] for q in pair_list], np.int32))
        cnt = len(pair_list)
        parts = pl.pallas_call(
            functools.partial(_sweep_body, n_valid=n, b=_BC,
                              masked=masked_flag),
            grid_spec=pltpu.PrefetchScalarGridSpec(
                num_scalar_prefetch=2,
                grid=(cnt,),
                in_specs=[
                    pl.BlockSpec((npad, hdim), lambda p, ii, jj: (0, 0)),
                    pl.BlockSpec((npad, hdim), lambda p, ii, jj: (0, 0)),
                    pl.BlockSpec((npad, hdim), lambda p, ii, jj: (0, 0)),
                    pl.BlockSpec((npad, hdim), lambda p, ii, jj: (0, 0)),
                    pl.BlockSpec((_BC, _BC), lambda p, ii, jj: (ii[p], jj[p])),
                    pl.BlockSpec((_BC, _BC), lambda p, ii, jj: (jj[p], ii[p])),
                    pl.BlockSpec((_BC, _BC), lambda p, ii, jj: (ii[p], jj[p])),
                    pl.BlockSpec((_BC, _BC), lambda p, ii, jj: (jj[p], ii[p])),
                ],
                out_specs=pl.BlockSpec((1, 16, _BC),
                                       lambda p, ii, jj: (p, 0, 0)),
            ),
            out_shape=jax.ShapeDtypeStruct((cnt, 16, _BC), jnp.float32),
            compiler_params=pltpu.CompilerParams(
                dimension_semantics=("arbitrary",)),
        )(ii, jj, amin, bmin, amax, bmax, pos, pos, pos_outer, pos_outer)
        return ii, jj, parts

    acc = jnp.zeros((t_blk, 8, _BC), jnp.float32)
    for plist, mflag in ((int_pairs, False), (edge_pairs, True)):
        if plist:
            ii, jj, parts = sweep(plist, mflag)
            acc = acc.at[ii].add(parts[:, 0:8, :]).at[jj].add(
                parts[:, 8:16, :])
    stats = acc.transpose(1, 0, 2).reshape(8, npad)[:, :n]
    rs_min, cs_min, rd_min, cd_min, rs_max, cs_max, rd_max, cd_max = stats

    eps = 1e-8
    ratios = jnp.stack([rd_min / (rs_min + eps), cd_min / (cs_min + eps),
                        rd_max / (rs_max + eps), cd_max / (cs_max + eps)])
    lori_mp, lori_sc, l1, l2 = _exact_means(jnp.log(ratios))
    loss_min = lam * (-lori_mp) + (1.0 - lam) * (-lori_sc)
    loss_max = (l1 + l2) / 2.0
    return loss_min + loss_max


# speedup vs baseline: 1.1366x; 1.0152x over previous
import functools

import numpy as np
import jax
import jax.numpy as jnp
from jax.experimental import pallas as pl
from jax.experimental.pallas import tpu as pltpu

_BM = 256
_BC = 640


def _exact_means(v):
    r, n = v.shape
    size = 1
    while size < n:
        size *= 2
    x = jnp.pad(v, ((0, 0), (0, size - n)))
    err_total = jnp.zeros((r,), jnp.float32)
    while size > 1:
        h = size // 2
        a, b = x[:, :h], x[:, h:]
        s = a + b
        bv = s - a
        err = (a - (s - bv)) + (b - bv)
        err_total = err_total + jnp.sum(err, axis=1)
        x = s
        size = h
    return (x[:, 0] + err_total) / n


def _gcn_body(adj_ref, fa_ref, wcat_ref, aw1t_ref, ab1_ref, aw2t_ref, avec_ref,
              z_ref, w_ref, seq_ref, *, n_valid, bm):
    m = pl.program_id(0)

    @pl.when(m == 0)
    def _init():
        seq_ref[...] = jnp.dot(fa_ref[...], wcat_ref[...],
                               preferred_element_type=jnp.float32
                               ).astype(jnp.bfloat16)
        w_ref[0] = 0.0
        w_ref[1] = 0.0

    adj_b = adj_ref[...].astype(jnp.bfloat16)
    h = jnp.dot(adj_b, seq_ref[...], preferred_element_type=jnp.float32)
    z = jnp.where(h >= 0.0, h, avec_ref[...] * h)
    z_ref[...] = z
    rows = m * bm + jax.lax.broadcasted_iota(jnp.int32, (bm, 1), 0)
    hh = z.shape[1] // 2
    for l in range(2):
        zl = z[:, l * hh:(l + 1) * hh]
        t = jnp.tanh(jnp.dot(zl, aw1t_ref[...],
                             preferred_element_type=jnp.float32) + ab1_ref[...])
        s = jnp.dot(t, aw2t_ref[...], preferred_element_type=jnp.float32)
        s = jnp.where(rows < n_valid, s, 0.0)
        w_ref[l] = w_ref[l] + jnp.sum(s)


def _proj_body(z_ref, f_ref, beta_ref, lwt_ref,
               mw1t_ref, mb1_ref, mw2t_ref, mb2_ref,
               xw1t_ref, xb1_ref, xw2t_ref, xb2_ref,
               amin_ref, bmin_ref, amax_ref, bmax_ref,
               *, n_valid, bc, inv_tau):
    hh = z_ref.shape[1] // 2
    z_mp = z_ref[:, :hh] * beta_ref[0] + z_ref[:, hh:] * beta_ref[1]
    z_sc = jnp.dot(f_ref[...], lwt_ref[...], preferred_element_type=jnp.float32)
    rows = pl.program_id(0) * bc + jax.lax.broadcasted_iota(
        jnp.int32, (bc, 1), 0)
    valid = rows < n_valid

    def proj(x, w1t, b1, w2t, b2, scale):
        t = jnp.dot(x, w1t, preferred_element_type=jnp.float32) + b1
        t = jnp.where(t > 0.0, t, jnp.exp(t) - 1.0)
        p = jnp.dot(t, w2t, preferred_element_type=jnp.float32) + b2
        nrm = jnp.sqrt(jnp.sum(p * p, axis=1, keepdims=True))
        p = p * (scale / jnp.maximum(nrm, 1e-12))
        return jnp.where(valid, p, 0.0).astype(jnp.bfloat16)

    amin_ref[...] = proj(z_mp, mw1t_ref[...], mb1_ref[...], mw2t_ref[...],
                         mb2_ref[...], inv_tau)
    bmin_ref[...] = proj(z_sc, mw1t_ref[...], mb1_ref[...], mw2t_ref[...],
                         mb2_ref[...], 1.0)
    amax_ref[...] = proj(z_mp, xw1t_ref[...], xb1_ref[...], xw2t_ref[...],
                         xb2_ref[...], inv_tau)
    bmax_ref[...] = proj(z_sc, xw1t_ref[...], xb1_ref[...], xw2t_ref[...],
                         xb2_ref[...], 1.0)


def _sweep_body(ii_ref, jj_ref, amin_ref, bmin_ref, amax_ref, bmax_ref,
                pij_ref, pji_ref, qij_ref, qji_ref, out_ref, *, n_valid, b,
                masked):
    p = pl.program_id(0)
    bi = ii_ref[p]
    bj = jj_ref[p]
    i0 = bi * b
    j0 = bj * b
    nd = jnp.where(bi == bj, 0.0, 1.0)
    if masked:
        rows = jax.lax.broadcasted_iota(jnp.int32, (b, b), 0)
        cols = jax.lax.broadcasted_iota(jnp.int32, (b, b), 1)
        v_ij = ((i0 + rows) < n_valid) & ((j0 + cols) < n_valid)
        v_ji = ((j0 + rows) < n_valid) & ((i0 + cols) < n_valid)
        inv_i = jnp.maximum(i0 + b - n_valid, 0).astype(jnp.float32)
        inv_j = jnp.maximum(j0 + b - n_valid, 0).astype(jnp.float32)
        pijT = jnp.where(v_ij, pij_ref[...], 0.0).T
        pjiT = jnp.where(v_ji, pji_ref[...], 0.0).T
        qijT = jnp.where(v_ij, qij_ref[...], 0.0).T
        qjiT = jnp.where(v_ji, qji_ref[...], 0.0).T
    else:
        inv_i = inv_j = 0.0
        pijT = pij_ref[...].T
        pjiT = pji_ref[...].T
        qijT = qij_ref[...].T
        qjiT = qji_ref[...].T

    def half(a, b2, pos_t_row, pos_t_col, inv_r, inv_c):
        s = jax.lax.dot_general(a, b2, (((1,), (1,)), ((), ())),
                                preferred_element_type=jnp.float32)
        st = jax.lax.dot_general(b2, a, (((1,), (1,)), ((), ())),
                                 preferred_element_type=jnp.float32)
        m = jnp.exp(s)
        mt = jnp.exp(st)
        rowsum = jnp.sum(mt, axis=0) - inv_c
        colsum = jnp.sum(m, axis=0) - inv_r
        rowdot = jnp.sum(mt * pos_t_row, axis=0)
        coldot = jnp.sum(m * pos_t_col, axis=0)
        return rowsum, colsum, rowdot, coldot

    for c, (a_ref, b_ref, dijT, djiT) in enumerate((
            (amin_ref, bmin_ref, pijT, pjiT),
            (amax_ref, bmax_ref, qijT, qjiT))):
        a_i = a_ref[pl.ds(i0, b), :]
        a_j = a_ref[pl.ds(j0, b), :]
        b_i = b_ref[pl.ds(i0, b), :]
        b_j = b_ref[pl.ds(j0, b), :]
        rsI, csJ, rdI, cdJ = half(a_i, b_j, dijT, djiT, inv_i, inv_j)
        rsJ, csI, rdJ, cdI = half(a_j, b_i, djiT, dijT, inv_j, inv_i)
        o = 4 * c
        out_ref[0, o + 0, :] = rsI
        out_ref[0, o + 1, :] = csI * nd
        out_ref[0, o + 2, :] = rdI
        out_ref[0, o + 3, :] = cdI * nd
        out_ref[0, o + 8, :] = rsJ * nd
        out_ref[0, o + 9, :] = csJ
        out_ref[0, o + 10, :] = rdJ * nd
        out_ref[0, o + 11, :] = cdJ


def kernel(adj, features, pos, pos_outer, gcn_W, prelu_a, att_W1, att_b1,
           att_W2, min_W1, min_b1, min_W2, min_b2, max_W1, max_b1, max_W2,
           max_b2, l_W):
    tau, lam = 0.8, 0.5
    n, d = features.shape
    hdim = gcn_W.shape[1]
    t_blk = pl.cdiv(n, _BC)
    npad = t_blk * _BC
    assert npad % _BM == 0
    m_blk = npad // _BM

    noise = jax.random.normal(jax.random.key(42), features.shape,
                              features.dtype) * 0.01
    nn = jnp.linalg.norm(noise, axis=1, keepdims=True)
    fa = features + noise / jnp.maximum(nn, 1e-12)
    wcat = jnp.concatenate([gcn_W[0].T, gcn_W[1].T], axis=1)
    avec = jnp.concatenate(
        [jnp.full((1, hdim), prelu_a[0], jnp.float32),
         jnp.full((1, hdim), prelu_a[1], jnp.float32)], axis=1)
    ab1 = att_b1.reshape(1, hdim)
    aw2t = att_W2.T

    z, wsum = pl.pallas_call(
        functools.partial(_gcn_body, n_valid=n, bm=_BM),
        grid=(m_blk,),
        in_specs=[
            pl.BlockSpec((_BM, n), lambda m: (m, 0)),
            pl.BlockSpec((n, d), lambda m: (0, 0)),
            pl.BlockSpec((d, 2 * hdim), lambda m: (0, 0)),
            pl.BlockSpec((hdim, hdim), lambda m: (0, 0)),
            pl.BlockSpec((1, hdim), lambda m: (0, 0)),
            pl.BlockSpec((hdim, 1), lambda m: (0, 0)),
            pl.BlockSpec((1, 2 * hdim), lambda m: (0, 0)),
        ],
        out_specs=[
            pl.BlockSpec((_BM, 2 * hdim), lambda m: (m, 0)),
            pl.BlockSpec(memory_space=pltpu.SMEM),
        ],
        out_shape=[
            jax.ShapeDtypeStruct((npad, 2 * hdim), jnp.float32),
            jax.ShapeDtypeStruct((2,), jnp.float32),
        ],
        scratch_shapes=[pltpu.VMEM((n, 2 * hdim), jnp.bfloat16)],
        compiler_params=pltpu.CompilerParams(
            dimension_semantics=("arbitrary",)),
    )(adj, fa, wcat, att_W1.T, ab1, aw2t, avec)

    beta = jax.nn.softmax(wsum / n)

    amin, bmin, amax, bmax = pl.pallas_call(
        functools.partial(_proj_body, n_valid=n, bc=_BC, inv_tau=1.0 / tau),
        grid=(t_blk,),
        in_specs=[
            pl.BlockSpec((_BC, 2 * hdim), lambda m: (m, 0)),
            pl.BlockSpec((_BC, d), lambda m: (m, 0)),
            pl.BlockSpec(memory_space=pltpu.SMEM),
            pl.BlockSpec((d, hdim), lambda m: (0, 0)),
            pl.BlockSpec((hdim, hdim), lambda m: (0, 0)),
            pl.BlockSpec((1, hdim), lambda m: (0, 0)),
            pl.BlockSpec((hdim, hdim), lambda m: (0, 0)),
            pl.BlockSpec((1, hdim), lambda m: (0, 0)),
            pl.BlockSpec((hdim, hdim), lambda m: (0, 0)),
            pl.BlockSpec((1, hdim), lambda m: (0, 0)),
            pl.BlockSpec((hdim, hdim), lambda m: (0, 0)),
            pl.BlockSpec((1, hdim), lambda m: (0, 0)),
        ],
        out_specs=[pl.BlockSpec((_BC, hdim), lambda m: (m, 0))] * 4,
        out_shape=[jax.ShapeDtypeStruct((npad, hdim), jnp.bfloat16)] * 4,
        compiler_params=pltpu.CompilerParams(
            dimension_semantics=("arbitrary",)),
    )(z, features, beta, l_W.T,
      min_W1.T, min_b1.reshape(1, hdim), min_W2.T, min_b2.reshape(1, hdim),
      max_W1.T, max_b1.reshape(1, hdim), max_W2.T, max_b2.reshape(1, hdim))

    all_pairs = [(i, j) for i in range(t_blk) for j in range(i, t_blk)]
    ragged = (n % _BC) != 0
    edge_pairs = [q for q in all_pairs
                  if ragged and (q[0] == t_blk - 1 or q[1] == t_blk - 1)]
    int_pairs = [q for q in all_pairs if q not in edge_pairs]

    def sweep(pair_list, masked_flag):
        ii_np = np.array([q[0] for q in pair_list], np.int32)
        jj_np = np.array([q[1] for q in pair_list], np.int32)
        ii = jnp.asarray(ii_np)
        jj = jnp.asarray(jj_np)
        cnt = len(pair_list)
        parts = pl.pallas_call(
            functools.partial(_sweep_body, n_valid=n, b=_BC,
                              masked=masked_flag),
            grid_spec=pltpu.PrefetchScalarGridSpec(
                num_scalar_prefetch=2,
                grid=(cnt,),
                in_specs=[
                    pl.BlockSpec((npad, hdim), lambda p, ii, jj: (0, 0)),
                    pl.BlockSpec((npad, hdim), lambda p, ii, jj: (0, 0)),
                    pl.BlockSpec((npad, hdim), lambda p, ii, jj: (0, 0)),
                    pl.BlockSpec((npad, hdim), lambda p, ii, jj: (0, 0)),
                    pl.BlockSpec((_BC, _BC), lambda p, ii, jj: (ii[p], jj[p])),
                    pl.BlockSpec((_BC, _BC), lambda p, ii, jj: (jj[p], ii[p])),
                    pl.BlockSpec((_BC, _BC), lambda p, ii, jj: (ii[p], jj[p])),
                    pl.BlockSpec((_BC, _BC), lambda p, ii, jj: (jj[p], ii[p])),
                ],
                out_specs=pl.BlockSpec((1, 16, _BC),
                                       lambda p, ii, jj: (p, 0, 0)),
            ),
            out_shape=jax.ShapeDtypeStruct((cnt, 16, _BC), jnp.float32),
            compiler_params=pltpu.CompilerParams(
                dimension_semantics=("arbitrary",)),
        )(ii, jj, amin, bmin, amax, bmax, pos, pos, pos_outer, pos_outer)
        ohi = jnp.asarray(np.eye(t_blk, dtype=np.float32)[ii_np])
        ohj = jnp.asarray(np.eye(t_blk, dtype=np.float32)[jj_np])
        flat = parts.reshape(cnt, 16 * _BC)
        return (ohi.T @ flat[:, :8 * _BC] + ohj.T @ flat[:, 8 * _BC:])

    acc = jnp.zeros((t_blk, 8 * _BC), jnp.float32)
    for plist, mflag in ((int_pairs, False), (edge_pairs, True)):
        if plist:
            acc = acc + sweep(plist, mflag)
    stats = acc.reshape(t_blk, 8, _BC).transpose(1, 0, 2).reshape(
        8, npad)[:, :n]
    rs_min, cs_min, rd_min, cd_min, rs_max, cs_max, rd_max, cd_max = stats

    eps = 1e-8
    ratios = jnp.stack([rd_min / (rs_min + eps), cd_min / (cs_min + eps),
                        rd_max / (rs_max + eps), cd_max / (cs_max + eps)])
    lori_mp, lori_sc, l1, l2 = _exact_means(jnp.log(ratios))
    loss_min = lam * (-lori_mp) + (1.0 - lam) * (-lori_sc)
    loss_max = (l1 + l2) / 2.0
    return loss_min + loss_max
